# baseline (speedup 1.0000x reference)
import functools

import jax
import jax.numpy as jnp
from jax.experimental import pallas as pl
from jax.experimental.pallas import tpu as pltpu

_TILE = 128
_ECHUNK = 2048
_CP = pltpu.CompilerParams(vmem_limit_bytes=100 * 1024 * 1024)


def _mlp_in(x, W1, b1, W2, b2):
    y = jnp.maximum(x @ W1 + b1, 0.0)
    return y @ W2 + b2


def _proj(x, Wg, aa):
    h = x @ Wg
    hs = h * aa[0:1, :]
    hd = h * aa[1:2, :]
    als0 = jnp.sum(hs[:, :64], axis=1, keepdims=True)
    als1 = jnp.sum(hs[:, 64:], axis=1, keepdims=True)
    ald0 = jnp.sum(hd[:, :64], axis=1, keepdims=True)
    ald1 = jnp.sum(hd[:, 64:], axis=1, keepdims=True)
    z = jnp.zeros_like(als0)
    al = jnp.concatenate([als0, als1, ald0, ald1, z, z, z, z], axis=1)
    return h, al


def _static_node_kernel(sf_ref, emb_ref, W1_ref, b1_ref, W2_ref, b2_ref,
                        Wg_ref, aa_ref, h_ref, al_ref):
    tile = sf_ref.shape[0]
    iota = jax.lax.broadcasted_iota(jnp.int32, (tile, 16), 1)
    parts = []
    for f in range(4):
        col = sf_ref[:, f:f + 1]
        oh = (col == iota).astype(jnp.float32)
        parts.append(oh @ emb_ref[f])
    se = jnp.concatenate(parts, axis=1)
    x = _mlp_in(se, W1_ref[...], b1_ref[...], W2_ref[...], b2_ref[...])
    h, al = _proj(x, Wg_ref[...], aa_ref[...])
    h_ref[...] = h
    al_ref[...] = al


def _dyn_node_kernel(df_ref, W1_ref, b1_ref, W2_ref, b2_ref,
                     Wg_ref, aa_ref, h_ref, al_ref):
    x = _mlp_in(df_ref[...], W1_ref[...], b1_ref[...], W2_ref[...], b2_ref[...])
    h, al = _proj(x, Wg_ref[...], aa_ref[...])
    h_ref[...] = h
    al_ref[...] = al


def _edge_kernel(e_ref, w_ref, al_ref, h_ref, num_ref, den_ref):
    @pl.when(pl.program_id(0) == 0)
    def _init():
        num_ref[...] = jnp.zeros_like(num_ref)
        den_ref[...] = jnp.zeros_like(den_ref)

    chunk = e_ref.shape[1]

    def body(i, _):
        s = e_ref[0, i]
        d = e_ref[1, i]
        w = w_ref[0, i]
        a_s = al_ref[pl.ds(s, 1), :]
        a_d = al_ref[pl.ds(d, 1), :]
        e2 = a_s[:, 0:2] + a_d[:, 2:4]
        e2 = jnp.where(e2 > 0, e2, 0.2 * e2)
        ex = jnp.exp(e2) * w
        hrow = h_ref[pl.ds(s, 1), :]
        exb = jnp.concatenate(
            [jnp.broadcast_to(ex[:, 0:1], (1, 64)),
             jnp.broadcast_to(ex[:, 1:2], (1, 64))], axis=1)
        num_ref[pl.ds(d, 1), :] = num_ref[pl.ds(d, 1), :] + hrow * exb
        den_ref[pl.ds(d, 1), :] = den_ref[pl.ds(d, 1), :] + exb
        return 0

    jax.lax.fori_loop(0, chunk, body, 0)


def _post_kernel(num_ref, den_ref, W3_ref, b3_ref, W4_ref, b4_ref, out_ref):
    num = num_ref[...]
    den = den_ref[...]
    d0 = den[:, 0:1] + 1e-16
    d1 = den[:, 64:65] + 1e-16
    o = jnp.concatenate([num[:, :64] / d0, num[:, 64:] / d1], axis=1)
    out_ref[...] = _mlp_in(o, W3_ref[...], b3_ref[...], W4_ref[...], b4_ref[...])


def _final_kernel(ids_ref, se_ref, de_ref, btf_ref,
                  Wq_ref, Wk_ref, Wv_ref, Wo_ref, Wt_ref, bt_ref,
                  f1_ref, fb1_ref, f2_ref, fb2_ref,
                  out_ref, rs_ref, rd_ref):
    nrow = out_ref.shape[0]
    lseq = nrow // 2

    def gath(j, _):
        r = ids_ref[0, j]
        dr = ids_ref[1, j]
        rs_ref[pl.ds(j, 1), :] = se_ref[pl.ds(r, 1), :]
        rd_ref[pl.ds(j, 1), :] = de_ref[pl.ds(dr, 1), :]
        return 0

    jax.lax.fori_loop(0, nrow, gath, 0)
    road = jnp.concatenate([rs_ref[...], rd_ref[...]], axis=1)
    q = road @ Wq_ref[...]
    k = road @ Wk_ref[...]
    v = road @ Wv_ref[...]
    scale = 1.0 / jnp.sqrt(jnp.float32(256.0))
    outs = []
    for b in range(2):
        qb = q[b * lseq:(b + 1) * lseq, :]
        kb = k[b * lseq:(b + 1) * lseq, :]
        vb = v[b * lseq:(b + 1) * lseq, :]
        sc = jax.lax.dot_general(qb, kb, (((1,), (1,)), ((), ()))) * scale
        sc = sc - jnp.max(sc, axis=1, keepdims=True)
        p = jnp.exp(sc)
        p = p / jnp.sum(p, axis=1, keepdims=True)
        outs.append(p @ vb)
    att = jnp.concatenate(outs, axis=0)
    road2 = att @ Wo_ref[...]
    te = btf_ref[...] @ Wt_ref[...] + bt_ref[...]
    emb = jnp.concatenate([road2, te], axis=1)
    out_ref[...] = _mlp_in(emb, f1_ref[...], fb1_ref[...], f2_ref[...], fb2_ref[...])


def _full(shape, dtype=jnp.float32):
    return pl.BlockSpec(shape, lambda *_: tuple(0 for _ in shape))


def _node_call(kfn, nrows, ins, in_specs):
    grid = pl.cdiv(nrows, _TILE)
    return pl.pallas_call(
        kfn,
        grid=(grid,),
        in_specs=in_specs,
        out_specs=[
            pl.BlockSpec((_TILE, 128), lambda i: (i, 0)),
            pl.BlockSpec((_TILE, 8), lambda i: (i, 0)),
        ],
        out_shape=[
            jax.ShapeDtypeStruct((nrows, 128), jnp.float32),
            jax.ShapeDtypeStruct((nrows, 8), jnp.float32),
        ],
        compiler_params=_CP,
    )(*ins)


def _edge_call(e_pad, w_pad, al, h):
    nrows = h.shape[0]
    grid = e_pad.shape[1] // _ECHUNK
    return pl.pallas_call(
        _edge_kernel,
        grid=(grid,),
        in_specs=[
            pl.BlockSpec((2, _ECHUNK), lambda i: (0, i),
                         memory_space=pltpu.SMEM),
            pl.BlockSpec((1, _ECHUNK), lambda i: (0, i),
                         memory_space=pltpu.SMEM),
            pl.BlockSpec((nrows, 8), lambda i: (0, 0)),
            pl.BlockSpec((nrows, 128), lambda i: (0, 0)),
        ],
        out_specs=[
            pl.BlockSpec((nrows, 128), lambda i: (0, 0)),
            pl.BlockSpec((nrows, 128), lambda i: (0, 0)),
        ],
        out_shape=[
            jax.ShapeDtypeStruct((nrows, 128), jnp.float32),
            jax.ShapeDtypeStruct((nrows, 128), jnp.float32),
        ],
        compiler_params=_CP,
    )(e_pad, w_pad, al, h)


def _post_call(num, den, W3, b3, W4, b4):
    nrows = num.shape[0]
    grid = pl.cdiv(nrows, _TILE)
    return pl.pallas_call(
        _post_kernel,
        grid=(grid,),
        in_specs=[
            pl.BlockSpec((_TILE, 128), lambda i: (i, 0)),
            pl.BlockSpec((_TILE, 128), lambda i: (i, 0)),
            _full((128, 128)), _full((1, 128)),
            _full((128, 128)), _full((1, 128)),
        ],
        out_specs=pl.BlockSpec((_TILE, 128), lambda i: (i, 0)),
        out_shape=jax.ShapeDtypeStruct((nrows, 128), jnp.float32),
        compiler_params=_CP,
    )(num, den, W3, b3, W4, b4)


def kernel(batch_road_id, batch_time_id, batch_time_features, static_features,
           edges, edge_weight, dynamic_features, emb_tables,
           sW1, sb1, sW2, sb2, sWg, sa_src, sa_dst, sW3, sb3, sW4, sb4,
           dW1, db1, dW2, db2, dWg, da_src, da_dst, dW3, db3, dW4, db4,
           Wq, Wk, Wv, Wo, Wt, bt, fW1, fb1, fW2, fb2):
    n1 = static_features.shape[0]
    n = n1 - 1
    m = edges.shape[1]
    bsz, lseq = batch_road_id.shape
    s_feat = batch_time_features.shape[-1]
    demb = sW2.shape[1]

    f32 = jnp.float32
    sWg_f = sWg.reshape(demb, 128)
    dWg_f = dWg.reshape(demb, 128)
    saa = jnp.stack([sa_src.reshape(-1), sa_dst.reshape(-1)], axis=0)
    daa = jnp.stack([da_src.reshape(-1), da_dst.reshape(-1)], axis=0)

    def row(v):
        return v.reshape(1, -1)

    sf = static_features.astype(jnp.int32)
    h_s, al_s = _node_call(
        _static_node_kernel, n1,
        [sf, emb_tables, sW1, row(sb1), sW2, row(sb2), sWg_f, saa],
        [
            pl.BlockSpec((_TILE, 4), lambda i: (i, 0)),
            _full(emb_tables.shape),
            _full(sW1.shape), _full((1, demb)),
            _full(sW2.shape), _full((1, demb)),
            _full((demb, 128)), _full((2, 128)),
        ])

    ep = pl.cdiv(m, _ECHUNK) * _ECHUNK
    e_pad = jnp.concatenate(
        [edges.astype(jnp.int32),
         jnp.zeros((2, ep - m), jnp.int32)], axis=1)
    w_pad = jnp.concatenate(
        [edge_weight.astype(f32), jnp.zeros((ep - m,), f32)]).reshape(1, ep)

    num_s, den_s = _edge_call(e_pad, w_pad, al_s, h_s)
    se2 = _post_call(num_s, den_s, sW3, row(sb3), sW4, row(sb4))

    tid0 = batch_time_id[:, 0]
    de = dynamic_features[:, tid0]
    de_flat = de.transpose(1, 0, 2).reshape(bsz * n1, s_feat)
    pad_cols = 8 - s_feat
    de_p = jnp.concatenate(
        [de_flat, jnp.zeros((bsz * n1, pad_cols), f32)], axis=1)
    dW1_p = jnp.concatenate([dW1, jnp.zeros((pad_cols, demb), f32)], axis=0)

    h_d, al_d = _node_call(
        _dyn_node_kernel, bsz * n1,
        [de_p, dW1_p, row(db1), dW2, row(db2), dWg_f, daa],
        [
            pl.BlockSpec((_TILE, 8), lambda i: (i, 0)),
            _full((8, demb)), _full((1, demb)),
            _full(dW2.shape), _full((1, demb)),
            _full((demb, 128)), _full((2, 128)),
        ])

    num_d = jnp.zeros((bsz * n1, 128), f32)
    den_d = jnp.zeros((bsz * n1, 128), f32)
    for b in range(bsz):
        lo = b * n
        h_w = jax.lax.dynamic_slice_in_dim(h_d, lo, n1, 0)
        al_w = jax.lax.dynamic_slice_in_dim(al_d, lo, n1, 0)
        nb, db_ = _edge_call(e_pad, w_pad, al_w, h_w)
        num_d = num_d.at[lo:lo + n1].add(nb)
        den_d = den_d.at[lo:lo + n1].add(db_)
    de2 = _post_call(num_d, den_d, dW3, row(db3), dW4, row(db4))

    rid = batch_road_id.astype(jnp.int32)
    ids = rid.reshape(-1)
    dids = (rid + (jnp.arange(bsz, dtype=jnp.int32) * n1)[:, None]).reshape(-1)
    ids2 = jnp.stack([ids, dids], axis=0)

    btf = batch_time_features.reshape(bsz * lseq, s_feat)
    btf_p = jnp.concatenate(
        [btf, jnp.zeros((bsz * lseq, pad_cols), f32)], axis=1)
    Wt_p = jnp.concatenate([Wt, jnp.zeros((pad_cols, demb), f32)], axis=0)

    nrow = bsz * lseq
    out = pl.pallas_call(
        _final_kernel,
        grid=(1,),
        in_specs=[
            pl.BlockSpec((2, nrow), lambda i: (0, 0),
                         memory_space=pltpu.SMEM),
            _full((n1, 128)),
            _full((bsz * n1, 128)),
            _full((nrow, 8)),
            _full(Wq.shape), _full(Wk.shape), _full(Wv.shape), _full(Wo.shape),
            _full((8, demb)), _full((1, demb)),
            _full(fW1.shape), _full((1, fW1.shape[1])),
            _full(fW2.shape), _full((1, fW2.shape[1])),
        ],
        out_specs=pl.BlockSpec((nrow, 256), lambda i: (0, 0)),
        out_shape=jax.ShapeDtypeStruct((nrow, 256), jnp.float32),
        scratch_shapes=[pltpu.VMEM((nrow, 128), f32),
                        pltpu.VMEM((nrow, 128), f32)],
        compiler_params=_CP,
    )(ids2, se2, de2, btf_p,
      Wq, Wk, Wv, Wo, Wt_p, row(bt),
      fW1, row(fb1), fW2, row(fb2))

    return out.reshape(bsz, lseq, 256)
